# baseline (speedup 1.0000x reference)
import functools

import jax
import jax.numpy as jnp
from jax import lax
from jax.experimental import pallas as pl
from jax.experimental.pallas import tpu as pltpu
from jax.experimental.pallas import tpu_sc as plsc

B = 1024
V = 100000
NC = 2
NS = 16
L = 16
IT = 128
NI = B // IT
NIC = NI // NC
NQ = NS // NIC
SPAN = V // NQ
WJ = 200
NCHK = SPAN // WJ
NBUF = 2
KG = IT // L
NEG_INF = float("-inf")

def _build(interpret=False):
  mesh = plsc.VectorSubcoreMesh(
      core_axis_name="c", subcore_axis_name="s", num_cores=NC, num_subcores=NS
  )

  @functools.partial(
    pl.kernel,
    out_type=jax.ShapeDtypeStruct((B,), jnp.float32),
    mesh=mesh,
    interpret=interpret,
    scratch_types=[
          [pltpu.VMEM((WJ, IT), jnp.float32) for _ in range(NBUF)],
          pltpu.VMEM((IT,), jnp.int32),
          pltpu.VMEM((IT,), jnp.float32),
          pltpu.VMEM((IT,), jnp.float32),
          pltpu.VMEM((IT,), jnp.float32),
          pltpu.VMEM((IT,), jnp.float32),
          pltpu.VMEM((IT,), jnp.float32),
          pltpu.VMEM_SHARED((NS, IT), jnp.float32),
          pltpu.VMEM_SHARED((NS, IT), jnp.float32),
          [pltpu.SemaphoreType.DMA] * NBUF,
      ],
      compiler_params=pltpu.CompilerParams(
          needs_layout_passes=False, use_tc_tiling_on_sc=True
      ),
  )
  def _cw_loss_sc(xt_hbm, y_hbm, out_hbm, buf, yv, accb, tvb, tmpm, tmpt,
                  lossb, spm, spt, sems):
      c = lax.axis_index("c")
      s = lax.axis_index("s")
      il = s % NIC
      q = s // NIC
      i0 = (c * NIC + il) * IT
      j00 = q * SPAN

      pltpu.sync_copy(y_hbm.at[pl.ds(i0, IT)], yv)

      lane = lax.iota(jnp.int32, L)
      neg = jnp.full((L,), NEG_INF, jnp.float32)
      yvecs = [yv[pl.ds(k * L, L)] for k in range(KG)]

      def chunk_src(ch):
          return xt_hbm.at[pl.ds(j00 + ch * WJ, WJ), pl.ds(i0, IT)]

      for ch in range(NBUF):
          pltpu.async_copy(chunk_src(ch), buf[ch], sems[ch])

      for k in range(KG):
          accb[pl.ds(k * L, L)] = neg
          tvb[pl.ds(k * L, L)] = neg

      def process(ch, b):
          j0c = j00 + ch * WJ
          for k in range(KG):
              ycol = yvecs[k] - j0c
              inb = (ycol >= 0) & (ycol < WJ)
              idxj = jnp.clip(ycol, 0, WJ - 1)
              idxi = k * L + lane
              g = plsc.load_gather(buf[b], [idxj, idxi], mask=inb)
              tvb[pl.ds(k * L, L)] = jnp.where(inb, g, tvb[pl.ds(k * L, L)])
              plsc.store_scatter(buf[b], [idxj, idxi], neg, mask=inb)

          def tile_body(jt, accs):
              out = list(accs)
              for jj in range(8):
                  for k in range(KG):
                      out[k] = jnp.maximum(
                          out[k], buf[b][jt * 8 + jj, pl.ds(k * L, L)]
                      )
              return tuple(out)

          accs = lax.fori_loop(
              0, WJ // 8, tile_body,
              tuple(accb[pl.ds(k * L, L)] for k in range(KG)),
          )
          for k in range(KG):
              accb[pl.ds(k * L, L)] = accs[k]

      def loop_body(it, _):
          for b in range(NBUF):
              ch = it * NBUF + b
              pltpu.make_async_copy(chunk_src(ch), buf[b], sems[b]).wait()
              process(ch, b)

              @pl.when(ch + NBUF < NCHK)
              def _():
                  pltpu.async_copy(chunk_src(ch + NBUF), buf[b], sems[b])

          return 0

      nfull = (NCHK // NBUF) * NBUF
      lax.fori_loop(0, NCHK // NBUF, loop_body, 0)
      for ch in range(nfull, NCHK):
          b = ch % NBUF
          pltpu.make_async_copy(chunk_src(ch), buf[b], sems[b]).wait()
          process(ch, b)

      pltpu.sync_copy(accb, spm.at[s])
      pltpu.sync_copy(tvb, spt.at[s])
      plsc.subcore_barrier()

      @pl.when(q == 0)
      def _():
          for qq in range(1, NQ):
              pltpu.sync_copy(spm.at[s + qq * NIC], tmpm)
              pltpu.sync_copy(spt.at[s + qq * NIC], tmpt)
              for k in range(KG):
                  sl = pl.ds(k * L, L)
                  accb[sl] = jnp.maximum(accb[sl], tmpm[sl])
                  tvb[sl] = jnp.maximum(tvb[sl], tmpt[sl])
          for k in range(KG):
              sl = pl.ds(k * L, L)
              lossb[sl] = accb[sl] - tvb[sl]
          pltpu.sync_copy(lossb, out_hbm.at[pl.ds(i0, IT)])

  return _cw_loss_sc


_impl = _build()


def kernel(x, y):
    return _impl(x.T, y.astype(jnp.int32))


# speedup vs baseline: 1.3270x; 1.3270x over previous
import functools

import jax
import jax.numpy as jnp
from jax import lax
from jax.experimental import pallas as pl
from jax.experimental.pallas import tpu as pltpu
from jax.experimental.pallas import tpu_sc as plsc

B = 1024
V = 100000
VS = 51200
VT = V - VS
BC = 800
TCOFF = VS // BC
NC = 2
NS = 16
L = 16
IT = 128
NI = B // IT
NIC = NI // NC
NQ = NS // NIC
SPAN = VS // NQ
WJ = 200
NCHK = SPAN // WJ
NBUF = 4
KG = IT // L
NEG_INF = float("-inf")


def _build(interpret=False):
  mesh = plsc.VectorSubcoreMesh(
      core_axis_name="c", subcore_axis_name="s", num_cores=NC, num_subcores=NS
  )

  @functools.partial(
    pl.kernel,
    out_type=(
        jax.ShapeDtypeStruct((B,), jnp.float32),
        jax.ShapeDtypeStruct((B,), jnp.float32),
    ),
    mesh=mesh,
    interpret=interpret,
    scratch_types=[
          [pltpu.VMEM((WJ, IT), jnp.float32) for _ in range(NBUF)],
          pltpu.VMEM((IT,), jnp.int32),
          pltpu.VMEM((IT,), jnp.float32),
          pltpu.VMEM((IT,), jnp.float32),
          pltpu.VMEM((IT,), jnp.float32),
          pltpu.VMEM((IT,), jnp.float32),
          pltpu.VMEM_SHARED((NS, IT), jnp.float32),
          pltpu.VMEM_SHARED((NS, IT), jnp.float32),
          [pltpu.SemaphoreType.DMA] * NBUF,
      ],
      compiler_params=pltpu.CompilerParams(
          needs_layout_passes=False, use_tc_tiling_on_sc=True
      ),
  )
  def _cw_loss_sc(xt_hbm, y_hbm, outm_hbm, outt_hbm, buf, yv, accb, tvb,
                  tmpm, tmpt, spm, spt, sems):
      c = lax.axis_index("c")
      s = lax.axis_index("s")
      il = s % NIC
      q = s // NIC
      i0 = (c * NIC + il) * IT
      j00 = q * SPAN

      pltpu.sync_copy(y_hbm.at[pl.ds(i0, IT)], yv)

      lane = lax.iota(jnp.int32, L)
      neg = jnp.full((L,), NEG_INF, jnp.float32)
      yvecs = [yv[pl.ds(k * L, L)] for k in range(KG)]

      def chunk_src(ch):
          return xt_hbm.at[pl.ds(j00 + ch * WJ, WJ), pl.ds(i0, IT)]

      for ch in range(NBUF):
          pltpu.async_copy(chunk_src(ch), buf[ch], sems[ch])

      for k in range(KG):
          accb[pl.ds(k * L, L)] = neg
          tvb[pl.ds(k * L, L)] = neg

      def process(ch, b):
          j0c = j00 + ch * WJ
          for k in range(KG):
              ycol = yvecs[k] - j0c
              inb = (ycol >= 0) & (ycol < WJ)
              idxj = jnp.clip(ycol, 0, WJ - 1)
              idxi = k * L + lane
              g = plsc.load_gather(buf[b], [idxj, idxi], mask=inb)
              tvb[pl.ds(k * L, L)] = jnp.where(inb, g, tvb[pl.ds(k * L, L)])
              plsc.store_scatter(buf[b], [idxj, idxi], neg, mask=inb)

          def tile_body(jt, accs):
              out = list(accs)
              for jj in range(8):
                  for k in range(KG):
                      out[k] = jnp.maximum(
                          out[k], buf[b][jt * 8 + jj, pl.ds(k * L, L)]
                      )
              return tuple(out)

          accs = lax.fori_loop(
              0, WJ // 8, tile_body,
              tuple(accb[pl.ds(k * L, L)] for k in range(KG)),
          )
          for k in range(KG):
              accb[pl.ds(k * L, L)] = accs[k]

      def loop_body(it, _):
          for b in range(NBUF):
              ch = it * NBUF + b
              pltpu.make_async_copy(chunk_src(ch), buf[b], sems[b]).wait()
              process(ch, b)

              @pl.when(ch + NBUF < NCHK)
              def _():
                  pltpu.async_copy(chunk_src(ch + NBUF), buf[b], sems[b])

          return 0

      nfull = (NCHK // NBUF) * NBUF
      lax.fori_loop(0, NCHK // NBUF, loop_body, 0)
      for ch in range(nfull, NCHK):
          b = ch % NBUF
          pltpu.make_async_copy(chunk_src(ch), buf[b], sems[b]).wait()
          process(ch, b)

      pltpu.sync_copy(accb, spm.at[s])
      pltpu.sync_copy(tvb, spt.at[s])
      plsc.subcore_barrier()

      @pl.when(q == 0)
      def _():
          for qq in range(1, NQ):
              pltpu.sync_copy(spm.at[s + qq * NIC], tmpm)
              pltpu.sync_copy(spt.at[s + qq * NIC], tmpt)
              for k in range(KG):
                  sl = pl.ds(k * L, L)
                  accb[sl] = jnp.maximum(accb[sl], tmpm[sl])
                  tvb[sl] = jnp.maximum(tvb[sl], tmpt[sl])
          pltpu.sync_copy(accb, outm_hbm.at[pl.ds(i0, IT)])
          pltpu.sync_copy(tvb, outt_hbm.at[pl.ds(i0, IT)])

  return _cw_loss_sc


_impl = _build()


def _tc_body(xb, yb, mo, to):
    i = pl.program_id(0)
    col0 = VS + i * BC
    colid = lax.broadcasted_iota(jnp.int32, (BC, 1), 0) + col0
    mask = colid == yb[...]
    xv = xb[...]
    m = jnp.max(jnp.where(mask, NEG_INF, xv), axis=0)
    t = jnp.max(jnp.where(mask, xv, NEG_INF), axis=0)

    @pl.when(i == 0)
    def _():
        mo[...] = jnp.full((B,), NEG_INF, jnp.float32)
        to[...] = jnp.full((B,), NEG_INF, jnp.float32)

    mo[...] = jnp.maximum(mo[...], m)
    to[...] = jnp.maximum(to[...], t)


_tc_partial = pl.pallas_call(
    _tc_body,
    grid=(VT // BC,),
    in_specs=[
        pl.BlockSpec((BC, B), lambda i: (TCOFF + i, 0)),
        pl.BlockSpec((B,), lambda i: (0,)),
    ],
    out_specs=(
        pl.BlockSpec((B,), lambda i: (0,)),
        pl.BlockSpec((B,), lambda i: (0,)),
    ),
    out_shape=(
        jax.ShapeDtypeStruct((B,), jnp.float32),
        jax.ShapeDtypeStruct((B,), jnp.float32),
    ),
)


def _comb_body(m1, t1, m2, t2, o):
    o[...] = jnp.maximum(m1[...], m2[...]) - jnp.maximum(t1[...], t2[...])


_combine = pl.pallas_call(
    _comb_body,
    out_shape=jax.ShapeDtypeStruct((B,), jnp.float32),
)


def kernel(x, y):
    xt = x.T
    y32 = y.astype(jnp.int32)
    ms, ts = _impl(xt, y32)
    mt, tt = _tc_partial(xt, y32)
    return _combine(ms, ts, mt, tt)
